# baseline (speedup 1.0000x reference)
import dataclasses
import functools

import jax
import jax.numpy as jnp
from jax import lax
from jax.experimental import pallas as pl
from jax.experimental.pallas import tpu as pltpu
from jax.experimental.pallas import tpu_sc as plsc

EMBED = 32
NUM_CORES = 2
NUM_SUBCORES = 16
NUM_WORKERS = NUM_CORES * NUM_SUBCORES
LANES = 16
IDX_CHUNK = 128


@functools.partial(jax.jit, static_argnames=("batch",))
def _gmf_sc(user2d, item2d, wb, user_table, item_table, batch):
    b_per_w = batch // NUM_WORKERS
    n_chunk = b_per_w // IDX_CHUNK
    mesh = plsc.VectorSubcoreMesh(core_axis_name="c", subcore_axis_name="s")

    cp = dataclasses.replace(
        pltpu.CompilerParams(),
        use_tc_tiling_on_sc=False,
        needs_layout_passes=False,
    )

    @functools.partial(
        pl.kernel,
        mesh=mesh,
        out_type=jax.ShapeDtypeStruct((batch,), jnp.float32),
        scratch_types=[
            pltpu.VMEM((n_chunk, IDX_CHUNK), jnp.int32),
            pltpu.VMEM((n_chunk, IDX_CHUNK), jnp.int32),
            pltpu.VMEM((b_per_w, EMBED), jnp.float32),
            pltpu.VMEM((b_per_w, EMBED), jnp.float32),
            pltpu.VMEM((64,), jnp.float32),
            pltpu.VMEM((b_per_w,), jnp.float32),
            pltpu.SemaphoreType.DMA,
            pltpu.SemaphoreType.DMA,
        ],
        compiler_params=cp,
    )
    def k(user_hbm, item_hbm, wb_hbm, utab_hbm, itab_hbm, out_hbm,
          uidx_v, iidx_v, urows_v, irows_v, wb_v, out_v, sem_u, sem_i):
        wid = lax.axis_index("s") * NUM_CORES + lax.axis_index("c")
        base = wid * b_per_w

        pltpu.sync_copy(user_hbm.at[pl.ds(wid * n_chunk, n_chunk)], uidx_v)
        pltpu.sync_copy(item_hbm.at[pl.ds(wid * n_chunk, n_chunk)], iidx_v)
        pltpu.sync_copy(wb_hbm, wb_v)

        copies = []
        for c in range(n_chunk):
            copies.append(pltpu.async_copy(
                utab_hbm.at[uidx_v.at[c]],
                urows_v.at[pl.ds(c * IDX_CHUNK, IDX_CHUNK)], sem_u))
            copies.append(pltpu.async_copy(
                itab_hbm.at[iidx_v.at[c]],
                irows_v.at[pl.ds(c * IDX_CHUNK, IDX_CHUNK)], sem_i))
        for cp_ in copies:
            cp_.wait()

        w_splat = [
            plsc.load_gather(wb_v, [jnp.full((LANES,), LANES + d, jnp.int32)])
            for d in range(EMBED)
        ]
        b_splat = plsc.load_gather(
            wb_v, [jnp.full((LANES,), LANES + EMBED, jnp.int32)])
        lane_iota = lax.iota(jnp.int32, LANES)

        @pl.loop(0, b_per_w, step=LANES)
        def _(r0):
            row_idx = lane_iota + r0
            acc = b_splat
            for d in range(EMBED):
                d_idx = jnp.full((LANES,), d, jnp.int32)
                ug = plsc.load_gather(urows_v, [row_idx, d_idx])
                ig = plsc.load_gather(irows_v, [row_idx, d_idx])
                acc = acc + ug * ig * w_splat[d]
            out_v[pl.ds(r0, LANES)] = acc

        pltpu.sync_copy(out_v, out_hbm.at[pl.ds(base, b_per_w)])

    return k(user2d, item2d, wb, user_table, item_table)


def kernel(user, item, user_table, item_table, W, b):
    batch = user.shape[0]
    user2d = user.astype(jnp.int32).reshape(-1, IDX_CHUNK)
    item2d = item.astype(jnp.int32).reshape(-1, IDX_CHUNK)
    wb = jnp.concatenate(
        [jnp.zeros((LANES,), jnp.float32),
         W.reshape(-1).astype(jnp.float32), b.astype(jnp.float32),
         jnp.zeros((64 - LANES - EMBED - 1,), jnp.float32)])
    return _gmf_sc(user2d, item2d, wb, user_table, item_table, batch)
